# Initial kernel scaffold; baseline (speedup 1.0000x reference)
import functools

import jax
import jax.numpy as jnp
from jax import lax
from jax.experimental import pallas as pl
from jax.experimental.pallas import tpu as pltpu
from jax.experimental.pallas import tpu_sc as plsc

D = 384
K = 8192
BK = 1024
CCW = (0.05, 0.25, 0.6)

NC = 2
NS = 16
NW = NC * NS


def _argmin_body(zb2_ref, znorm_ref, cb_ref, cbnorm_ref, idx_ref, part_ref,
                 minval, minidx):
    k = pl.program_id(1)
    nk = pl.num_programs(1)
    cbb = cb_ref[...].astype(jnp.bfloat16)
    m2 = lax.dot_general(zb2_ref[...], cbb, (((1,), (1,)), ((), ())),
                         preferred_element_type=jnp.float32)
    t = znorm_ref[...] + m2
    dist = t + cbnorm_ref[...]
    m = jnp.min(dist, axis=1, keepdims=True)
    ids = lax.broadcasted_iota(jnp.int32, dist.shape, 1)
    loc = jnp.min(jnp.where(dist == m, ids, K), axis=1, keepdims=True)
    gid = loc + k * BK

    @pl.when(k == 0)
    def _():
        minval[...] = m
        minidx[...] = gid

    @pl.when(k > 0)
    def _():
        better = m < minval[...]
        minval[...] = jnp.where(better, m, minval[...])
        minidx[...] = jnp.where(better, gid, minidx[...])

    @pl.when(k == nk - 1)
    def _():
        idx_ref[...] = minidx[...]
        part_ref[0, 0] = jnp.sum(minval[...])


def _argmin_call(zb2, znorm, cb, cbnorm, bn, interpret=False):
    n = zb2.shape[0]
    nrb = n // bn
    nk = K // BK
    return pl.pallas_call(
        _argmin_body,
        grid=(nrb, nk),
        in_specs=[
            pl.BlockSpec((bn, D), lambda r, k: (r, 0)),
            pl.BlockSpec((bn, 1), lambda r, k: (r, 0)),
            pl.BlockSpec((BK, D), lambda r, k: (k, 0)),
            pl.BlockSpec((1, BK), lambda r, k: (0, k)),
        ],
        out_specs=[
            pl.BlockSpec((bn, 1), lambda r, k: (r, 0)),
            pl.BlockSpec((1, 1), lambda r, k: (r, 0)),
        ],
        out_shape=[
            jax.ShapeDtypeStruct((n, 1), jnp.int32),
            jax.ShapeDtypeStruct((nrb, 1), jnp.float32),
        ],
        scratch_shapes=[
            pltpu.VMEM((bn, 1), jnp.float32),
            pltpu.VMEM((bn, 1), jnp.int32),
        ],
        interpret=interpret,
    )(zb2, znorm, cb, cbnorm)


@functools.lru_cache(maxsize=None)
def _make_gather(n):
    b_per_w = n // NW
    mesh = plsc.VectorSubcoreMesh(core_axis_name="c", subcore_axis_name="s")

    @functools.partial(
        pl.kernel,
        mesh=mesh,
        out_type=jax.ShapeDtypeStruct((n, D), jnp.float32),
        scratch_types=[
            pltpu.VMEM((b_per_w,), jnp.int32),
            pltpu.VMEM((b_per_w, D), jnp.float32),
            pltpu.SemaphoreType.DMA,
        ],
    )
    def gather(cb_hbm, idx_hbm, out_hbm, idx_v, rows_v, sem):
        wid = lax.axis_index("s") * NC + lax.axis_index("c")
        base = wid * b_per_w
        pltpu.sync_copy(idx_hbm.at[pl.ds(base, b_per_w)], idx_v)
        pltpu.async_copy(cb_hbm.at[idx_v], rows_v, sem).wait()
        pltpu.sync_copy(rows_v, out_hbm.at[pl.ds(base, b_per_w)])

    return gather


def kernel(l0, l1, l2, cb0, cb1, cb2):
    out = []
    for i, (z, cb, bn) in enumerate(((l0, cb0, 1024), (l1, cb1, 2048),
                                     (l2, cb2, 2048))):
        flat = z.reshape(-1, D)
        n = flat.shape[0]
        znorm = jnp.sum(flat ** 2, axis=1, keepdims=True)
        cbnorm = jnp.sum(cb ** 2, axis=1)[None, :]
        zb2 = (-2.0 * flat).astype(jnp.bfloat16)
        idx2d, part = _argmin_call(zb2, znorm, cb, cbnorm, bn)
        idx = idx2d.reshape(z.shape[:-1])
        q = _make_gather(n)(cb, idx2d.reshape(-1)).reshape(z.shape)
        loss = jnp.float32(CCW[i]) * (jnp.sum(part) / jnp.float32(n * D))
        out.append((idx, loss, q))
    (idx0, loss0, q0), (idx1, loss1, q1), (idx2_, loss2, q2) = out
    total = loss0 + loss1 + loss2
    return (idx0, idx1, idx2_, total, q0, q1, q2)


# speedup vs baseline: 1.1885x; 1.1885x over previous
import functools

import jax
import jax.numpy as jnp
from jax import lax
from jax.experimental import pallas as pl
from jax.experimental.pallas import tpu as pltpu
from jax.experimental.pallas import tpu_sc as plsc

D = 384
K = 8192
BK = 1024
CCW = (0.05, 0.25, 0.6)

NC = 2
NS = 16
NW = NC * NS

_DOT_DTYPE = jnp.bfloat16


def _argmin_body(zb2_ref, znorm_ref, cb_ref, cbnorm_ref, idx_ref, part_ref,
                 minval, minidx):
    k = pl.program_id(1)
    nk = pl.num_programs(1)
    cbb = cb_ref[...].astype(_DOT_DTYPE)
    m2 = lax.dot_general(zb2_ref[...], cbb, (((1,), (1,)), ((), ())),
                         preferred_element_type=jnp.float32)
    t = znorm_ref[...] + m2
    dist = t + cbnorm_ref[...]
    m = jnp.min(dist, axis=1, keepdims=True)
    ids = lax.broadcasted_iota(jnp.int32, dist.shape, 1)
    loc = jnp.min(jnp.where(dist == m, ids, K), axis=1, keepdims=True)
    gid = loc + k * BK

    @pl.when(k == 0)
    def _():
        minval[...] = m
        minidx[...] = gid

    @pl.when(k > 0)
    def _():
        better = m < minval[...]
        minval[...] = jnp.where(better, m, minval[...])
        minidx[...] = jnp.where(better, gid, minidx[...])

    @pl.when(k == nk - 1)
    def _():
        idx_ref[...] = minidx[...]
        part_ref[...] = jnp.sum(minval[...], keepdims=True)[None]


def _argmin_call(zb2, znorm, cb, cbnorm, bn, interpret=False):
    n = zb2.shape[0]
    nrb = n // bn
    nk = K // BK
    return pl.pallas_call(
        _argmin_body,
        grid=(nrb, nk),
        in_specs=[
            pl.BlockSpec((bn, D), lambda r, k: (r, 0)),
            pl.BlockSpec((bn, 1), lambda r, k: (r, 0)),
            pl.BlockSpec((BK, D), lambda r, k: (k, 0)),
            pl.BlockSpec((1, BK), lambda r, k: (0, k)),
        ],
        out_specs=[
            pl.BlockSpec((bn, 1), lambda r, k: (r, 0)),
            pl.BlockSpec((1, 1, 1), lambda r, k: (r, 0, 0)),
        ],
        out_shape=[
            jax.ShapeDtypeStruct((n, 1), jnp.int32),
            jax.ShapeDtypeStruct((nrb, 1, 1), jnp.float32),
        ],
        scratch_shapes=[
            pltpu.VMEM((bn, 1), jnp.float32),
            pltpu.VMEM((bn, 1), jnp.int32),
        ],
        interpret=interpret,
    )(zb2, znorm, cb, cbnorm)


@functools.lru_cache(maxsize=None)
def _make_gather(n):
    b_per_w = n // NW
    mesh = plsc.VectorSubcoreMesh(core_axis_name="c", subcore_axis_name="s")

    @functools.partial(
        pl.kernel,
        mesh=mesh,
        out_type=jax.ShapeDtypeStruct((n, D), jnp.float32),
        scratch_types=[
            pltpu.VMEM((b_per_w,), jnp.int32),
            pltpu.VMEM((b_per_w, D), jnp.float32),
            pltpu.SemaphoreType.DMA,
        ],
    )
    def gather(cb_hbm, idx_hbm, out_hbm, idx_v, rows_v, sem):
        wid = lax.axis_index("s") * NC + lax.axis_index("c")
        base = wid * b_per_w
        pltpu.sync_copy(idx_hbm.at[pl.ds(base, b_per_w)], idx_v)
        pltpu.async_copy(cb_hbm.at[idx_v], rows_v, sem).wait()
        pltpu.sync_copy(rows_v, out_hbm.at[pl.ds(base, b_per_w)])

    return gather


def kernel(l0, l1, l2, cb0, cb1, cb2):
    out = []
    for i, (z, cb, bn) in enumerate(((l0, cb0, 1024), (l1, cb1, 2048),
                                     (l2, cb2, 2048))):
        flat = z.reshape(-1, D)
        n = flat.shape[0]
        znorm = jnp.sum(flat ** 2, axis=1, keepdims=True)
        cbnorm = jnp.sum(cb ** 2, axis=1)[None, :]
        zb2 = (-2.0 * flat).astype(_DOT_DTYPE)
        idx2d, part = _argmin_call(zb2, znorm, cb, cbnorm, bn)
        idx = idx2d.reshape(z.shape[:-1])
        q = _make_gather(n)(cb, idx2d.reshape(-1)).reshape(z.shape)
        loss = jnp.float32(CCW[i]) * (jnp.sum(part) / jnp.float32(n * D))
        out.append((idx, loss, q))
    (idx0, loss0, q0), (idx1, loss1, q1), (idx2_, loss2, q2) = out
    total = loss0 + loss1 + loss2
    return (idx0, idx1, idx2_, total, q0, q1, q2)
